# baseline (speedup 1.0000x reference)
import functools

import jax
import jax.numpy as jnp
from jax import lax
from jax.experimental import pallas as pl
from jax.experimental.pallas import tpu as pltpu
from jax.experimental.pallas import tpu_sc as plsc

_U = 1000000
_I = 1000000
_NE = 8
_D = 32
_B = 16384

_NC = 2
_NS = 16
_NW = _NC * _NS
_BW = _B // _NW
_CH = 128
_NCH = _BW // _CH
_G = _BW // 16

_LN2 = 0.6931471805599453
_SQRT2 = 1.4142135623730951


def _ln(x):
    xi = lax.bitcast_convert_type(x, jnp.int32)
    e = lax.shift_right_arithmetic(xi, 23) - 127
    m = lax.bitcast_convert_type(
        jnp.bitwise_or(jnp.bitwise_and(xi, 0x007FFFFF), 0x3F800000),
        jnp.float32)
    big = m > _SQRT2
    m = jnp.where(big, m * 0.5, m)
    e = e + jnp.where(big, 1, 0)
    t = (m - 1.0) / (m + 1.0)
    t2 = t * t
    ln_m = t * (2.0 + t2 * (2.0 / 3.0 + t2 * (2.0 / 5.0 + t2 * (2.0 / 7.0))))
    return e.astype(jnp.float32) * _LN2 + ln_m


def _body(users_hbm, items_hbm, envs_hbm,
          ue_inv_hbm, ub_inv_hbm, ie_inv_hbm, ib_inv_hbm,
          ue_env_hbm, ub_env_hbm, ie_env_hbm, ib_env_hbm,
          env_emb_hbm, env_bias_hbm, cls_w_hbm, cls_b_hbm,
          inv_out_hbm, env_out_hbm, eo_out_hbm,
          uidx_v, iidx_v, eidx_v,
          ru_inv, ri_inv, ru_env, ri_env,
          bu_inv, bi_inv, bu_env, bi_env,
          envemb_v, envbias_v, w_v, b_v,
          invs_v, envs_v, eo_v, sem):
    wid = lax.axis_index("s") * _NC + lax.axis_index("c")
    base = wid * _BW

    pltpu.sync_copy(users_hbm.at[pl.ds(base, _BW)], uidx_v)
    pltpu.sync_copy(items_hbm.at[pl.ds(base, _BW)], iidx_v)
    pltpu.sync_copy(envs_hbm.at[pl.ds(base, _BW)], eidx_v)

    copies = []
    for j in range(_NCH):
        s = pl.ds(j * _CH, _CH)
        for tab, idx, dst in (
                (ue_inv_hbm, uidx_v, ru_inv), (ie_inv_hbm, iidx_v, ri_inv),
                (ue_env_hbm, uidx_v, ru_env), (ie_env_hbm, iidx_v, ri_env)):
            copies.append(pltpu.async_copy(tab.at[idx.at[s]], dst.at[s], sem))
        for tab, idx, dst in (
                (ub_inv_hbm, uidx_v, bu_inv), (ib_inv_hbm, iidx_v, bi_inv),
                (ub_env_hbm, uidx_v, bu_env), (ib_env_hbm, iidx_v, bi_env)):
            copies.append(pltpu.async_copy(tab.at[idx.at[s]], dst.at[s], sem))

    pltpu.sync_copy(env_emb_hbm, envemb_v)
    pltpu.sync_copy(env_bias_hbm, envbias_v)
    pltpu.sync_copy(cls_w_hbm, w_v)
    pltpu.sync_copy(cls_b_hbm, b_v)

    for c in copies:
        c.wait()

    lane = lax.iota(jnp.int32, 16)
    w_rows = [[w_v[e, pl.ds(0, 16)], w_v[e, pl.ds(16, 16)]]
              for e in range(_NE)]
    b_vec = plsc.load_gather(b_v, [jnp.bitwise_and(lane, _NE - 1)])

    def group(g, carry):
        b16 = g * 16 + lane
        eids = eidx_v[pl.ds(g * 16, 16)]

        inv_acc = bu_inv[pl.ds(g * 16, 16)] + bi_inv[pl.ds(g * 16, 16)]
        env_acc = (bu_env[pl.ds(g * 16, 16)] + bi_env[pl.ds(g * 16, 16)]
                   + plsc.load_gather(envbias_v, [eids]))
        logits = [None] * _NE
        for d in range(_D):
            dd = jnp.full((16,), d, jnp.int32)
            u = plsc.load_gather(ru_inv, [b16, dd])
            i = plsc.load_gather(ri_inv, [b16, dd])
            p = u * i
            inv_acc = inv_acc + p
            ue = plsc.load_gather(ru_env, [b16, dd])
            ie = plsc.load_gather(ri_env, [b16, dd])
            ee = plsc.load_gather(envemb_v, [eids, dd])
            env_acc = env_acc + ue * ie * ee
            k, r = divmod(d, 16)
            for e in range(_NE):
                pw = p * w_rows[e][k][r]
                logits[e] = pw if logits[e] is None else logits[e] + pw

        logits = [logits[e] + b_vec[e] for e in range(_NE)]
        m = logits[0]
        for e in range(1, _NE):
            m = jnp.maximum(m, logits[e])
        exps = [jnp.exp(logits[e] - m) for e in range(_NE)]
        ssum = exps[0]
        for e in range(1, _NE):
            ssum = ssum + exps[e]
        lse = m + _ln(ssum)

        invs_v[pl.ds(g * 16, 16)] = inv_acc
        envs_v[pl.ds(g * 16, 16)] = inv_acc + env_acc
        for e in range(_NE):
            plsc.store_scatter(eo_v, [b16, jnp.full((16,), e, jnp.int32)],
                               logits[e] - lse)
        return carry

    lax.fori_loop(0, _G, group, 0)

    pltpu.sync_copy(invs_v, inv_out_hbm.at[pl.ds(base, _BW)])
    pltpu.sync_copy(envs_v, env_out_hbm.at[pl.ds(base, _BW)])
    pltpu.sync_copy(eo_v, eo_out_hbm.at[pl.ds(base, _BW)])


_TBK = 2048


def _tr_body(src_ref, dst_ref):
    dst_ref[...] = src_ref[...].T


@jax.jit
def _linearize(table_t):
    return pl.pallas_call(
        _tr_body,
        grid=(pl.cdiv(_U, _TBK),),
        in_specs=[pl.BlockSpec((_D, _TBK), lambda j: (0, j))],
        out_specs=pl.BlockSpec((_TBK, _D), lambda j: (j, 0)),
        out_shape=jax.ShapeDtypeStruct((_U, _D), jnp.float32),
    )(table_t)


@jax.jit
def _run(users_id, items_id, envs_id,
         ue_inv, ub_inv, ie_inv, ib_inv,
         ue_env, ub_env, ie_env, ib_env,
         env_emb, env_bias, cls_w, cls_b):
    mesh = plsc.VectorSubcoreMesh(core_axis_name="c", subcore_axis_name="s",
                                  num_cores=_NC, num_subcores=_NS)
    f32 = jnp.float32
    kernel_fn = pl.kernel(
        _body,
        out_type=(jax.ShapeDtypeStruct((_B,), f32),
                  jax.ShapeDtypeStruct((_B,), f32),
                  jax.ShapeDtypeStruct((_B, _NE), f32)),
        mesh=mesh,
        compiler_params=pltpu.CompilerParams(needs_layout_passes=False,
                                             use_tc_tiling_on_sc=False),
        scratch_types=[
            pltpu.VMEM((_BW,), jnp.int32),
            pltpu.VMEM((_BW,), jnp.int32),
            pltpu.VMEM((_BW,), jnp.int32),
            pltpu.VMEM((_BW, _D), f32),
            pltpu.VMEM((_BW, _D), f32),
            pltpu.VMEM((_BW, _D), f32),
            pltpu.VMEM((_BW, _D), f32),
            pltpu.VMEM((_BW,), f32),
            pltpu.VMEM((_BW,), f32),
            pltpu.VMEM((_BW,), f32),
            pltpu.VMEM((_BW,), f32),
            pltpu.VMEM((_NE, _D), f32),
            pltpu.VMEM((16,), f32),
            pltpu.VMEM((_NE, _D), f32),
            pltpu.VMEM((16,), f32),
            pltpu.VMEM((_BW,), f32),
            pltpu.VMEM((_BW,), f32),
            pltpu.VMEM((_BW, _NE), f32),
            pltpu.SemaphoreType.DMA,
        ],
    )
    return kernel_fn(users_id, items_id, envs_id,
                     ue_inv, ub_inv, ie_inv, ib_inv,
                     ue_env, ub_env, ie_env, ib_env,
                     env_emb, env_bias, cls_w, cls_b)


def kernel(users_id, items_id, envs_id, alpha,
           user_emb_inv, user_bias_inv, item_emb_inv, item_bias_inv,
           user_emb_env, user_bias_env, item_emb_env, item_bias_env,
           env_emb, env_bias, cls_W, cls_b):
    del alpha
    inv_score, env_score, env_outputs = _run(
        users_id, items_id, envs_id,
        _linearize(user_emb_inv.T), user_bias_inv.reshape(_U),
        _linearize(item_emb_inv.T), item_bias_inv.reshape(_I),
        _linearize(user_emb_env.T), user_bias_env.reshape(_U),
        _linearize(item_emb_env.T), item_bias_env.reshape(_I),
        env_emb, jnp.pad(env_bias.reshape(_NE), (0, 16 - _NE)),
        cls_W, jnp.pad(cls_b, (0, 16 - _NE)))
    return inv_score, env_score, env_outputs


# speedup vs baseline: 1.1645x; 1.1645x over previous
import functools

import jax
import jax.numpy as jnp
from jax import lax
from jax.experimental import pallas as pl
from jax.experimental.pallas import tpu as pltpu
from jax.experimental.pallas import tpu_sc as plsc

_U = 1000000
_I = 1000000
_NE = 8
_D = 32
_B = 16384

_NC = 2
_NS = 16
_NW = _NC * _NS
_BW = _B // _NW
_CH = 128
_NCH = _BW // _CH
_G = _BW // 16

_LN2 = 0.6931471805599453
_SQRT2 = 1.4142135623730951


def _ln(x):
    xi = lax.bitcast_convert_type(x, jnp.int32)
    e = lax.shift_right_arithmetic(xi, 23) - 127
    m = lax.bitcast_convert_type(
        jnp.bitwise_or(jnp.bitwise_and(xi, 0x007FFFFF), 0x3F800000),
        jnp.float32)
    big = m > _SQRT2
    m = jnp.where(big, m * 0.5, m)
    e = e + jnp.where(big, 1, 0)
    t = (m - 1.0) / (m + 1.0)
    t2 = t * t
    ln_m = t * (2.0 + t2 * (2.0 / 3.0 + t2 * (2.0 / 5.0 + t2 * (2.0 / 7.0))))
    return e.astype(jnp.float32) * _LN2 + ln_m


def _body(users_hbm, items_hbm, envs_hbm,
          ue_inv_hbm, ub_inv_hbm, ie_inv_hbm, ib_inv_hbm,
          ue_env_hbm, ub_env_hbm, ie_env_hbm, ib_env_hbm,
          env_emb_hbm, env_bias_hbm, cls_w_hbm, cls_b_hbm,
          inv_out_hbm, env_out_hbm, eo_out_hbm,
          uidx_v, iidx_v, eidx_v,
          ru_inv, ri_inv, ru_env, ri_env,
          bu_inv, bi_inv, bu_env, bi_env,
          envemb_v, envbias_v, w_v, b_v,
          invs_v, envs_v, eo_v, sem):
    wid = lax.axis_index("s") * _NC + lax.axis_index("c")
    base = wid * _BW

    pltpu.sync_copy(users_hbm.at[pl.ds(base, _BW)], uidx_v)
    pltpu.sync_copy(items_hbm.at[pl.ds(base, _BW)], iidx_v)
    pltpu.sync_copy(envs_hbm.at[pl.ds(base, _BW)], eidx_v)

    copies = []
    for j in range(_NCH):
        s = pl.ds(j * _CH, _CH)
        for tab, idx, dst in (
                (ue_inv_hbm, uidx_v, ru_inv), (ie_inv_hbm, iidx_v, ri_inv),
                (ue_env_hbm, uidx_v, ru_env), (ie_env_hbm, iidx_v, ri_env)):
            copies.append(pltpu.async_copy(tab.at[idx.at[s]], dst.at[s], sem))
        for tab, idx, dst in (
                (ub_inv_hbm, uidx_v, bu_inv), (ib_inv_hbm, iidx_v, bi_inv),
                (ub_env_hbm, uidx_v, bu_env), (ib_env_hbm, iidx_v, bi_env)):
            copies.append(pltpu.async_copy(tab.at[idx.at[s]], dst.at[s], sem))

    pltpu.sync_copy(env_emb_hbm, envemb_v)
    pltpu.sync_copy(env_bias_hbm, envbias_v)
    pltpu.sync_copy(cls_w_hbm, w_v)
    pltpu.sync_copy(cls_b_hbm, b_v)

    for c in copies:
        c.wait()

    lane = lax.iota(jnp.int32, 16)
    w_rows = [[w_v[e, pl.ds(0, 16)], w_v[e, pl.ds(16, 16)]]
              for e in range(_NE)]
    b_vec = plsc.load_gather(b_v, [jnp.bitwise_and(lane, _NE - 1)])

    def group(g, carry):
        b16 = g * 16 + lane
        eids = eidx_v[pl.ds(g * 16, 16)]

        inv_acc = bu_inv[pl.ds(g * 16, 16)] + bi_inv[pl.ds(g * 16, 16)]
        env_acc = (bu_env[pl.ds(g * 16, 16)] + bi_env[pl.ds(g * 16, 16)]
                   + plsc.load_gather(envbias_v, [eids]))
        logits = [None] * _NE
        for d in range(_D):
            dd = jnp.full((16,), d, jnp.int32)
            u = plsc.load_gather(ru_inv, [b16, dd])
            i = plsc.load_gather(ri_inv, [b16, dd])
            p = u * i
            inv_acc = inv_acc + p
            ue = plsc.load_gather(ru_env, [b16, dd])
            ie = plsc.load_gather(ri_env, [b16, dd])
            ee = plsc.load_gather(envemb_v, [eids, dd])
            env_acc = env_acc + ue * ie * ee
            k, r = divmod(d, 16)
            for e in range(_NE):
                pw = p * w_rows[e][k][r]
                logits[e] = pw if logits[e] is None else logits[e] + pw

        logits = [logits[e] + b_vec[e] for e in range(_NE)]
        m = logits[0]
        for e in range(1, _NE):
            m = jnp.maximum(m, logits[e])
        exps = [jnp.exp(logits[e] - m) for e in range(_NE)]
        ssum = exps[0]
        for e in range(1, _NE):
            ssum = ssum + exps[e]
        lse = m + _ln(ssum)

        invs_v[pl.ds(g * 16, 16)] = inv_acc
        envs_v[pl.ds(g * 16, 16)] = inv_acc + env_acc
        for e in range(_NE):
            plsc.store_scatter(eo_v, [b16, jnp.full((16,), e, jnp.int32)],
                               logits[e] - lse)
        return carry

    lax.fori_loop(0, _G, group, 0)

    pltpu.sync_copy(invs_v, inv_out_hbm.at[pl.ds(base, _BW)])
    pltpu.sync_copy(envs_v, env_out_hbm.at[pl.ds(base, _BW)])
    pltpu.sync_copy(eo_v, eo_out_hbm.at[pl.ds(base, _BW)])


_TBK = 4096


def _tr_body(src_ref, dst_ref):
    row = lax.broadcasted_iota(jnp.int32, (_D, _D), 0)
    col = lax.broadcasted_iota(jnp.int32, (_D, _D), 1)
    eye = (row == col).astype(jnp.float32)
    dst_ref[...] = lax.dot_general(
        src_ref[...], eye, (((0,), (0,)), ((), ())),
        preferred_element_type=jnp.float32)


@jax.jit
def _linearize(table_t):
    return pl.pallas_call(
        _tr_body,
        grid=(pl.cdiv(_U, _TBK),),
        in_specs=[pl.BlockSpec((_D, _TBK), lambda j: (0, j))],
        out_specs=pl.BlockSpec((_TBK, _D), lambda j: (j, 0)),
        out_shape=jax.ShapeDtypeStruct((_U, _D), jnp.float32),
    )(table_t)


@jax.jit
def _run(users_id, items_id, envs_id,
         ue_inv, ub_inv, ie_inv, ib_inv,
         ue_env, ub_env, ie_env, ib_env,
         env_emb, env_bias, cls_w, cls_b):
    mesh = plsc.VectorSubcoreMesh(core_axis_name="c", subcore_axis_name="s",
                                  num_cores=_NC, num_subcores=_NS)
    f32 = jnp.float32
    kernel_fn = pl.kernel(
        _body,
        out_type=(jax.ShapeDtypeStruct((_B,), f32),
                  jax.ShapeDtypeStruct((_B,), f32),
                  jax.ShapeDtypeStruct((_B, _NE), f32)),
        mesh=mesh,
        compiler_params=pltpu.CompilerParams(needs_layout_passes=False,
                                             use_tc_tiling_on_sc=False),
        scratch_types=[
            pltpu.VMEM((_BW,), jnp.int32),
            pltpu.VMEM((_BW,), jnp.int32),
            pltpu.VMEM((_BW,), jnp.int32),
            pltpu.VMEM((_BW, _D), f32),
            pltpu.VMEM((_BW, _D), f32),
            pltpu.VMEM((_BW, _D), f32),
            pltpu.VMEM((_BW, _D), f32),
            pltpu.VMEM((_BW,), f32),
            pltpu.VMEM((_BW,), f32),
            pltpu.VMEM((_BW,), f32),
            pltpu.VMEM((_BW,), f32),
            pltpu.VMEM((_NE, _D), f32),
            pltpu.VMEM((16,), f32),
            pltpu.VMEM((_NE, _D), f32),
            pltpu.VMEM((16,), f32),
            pltpu.VMEM((_BW,), f32),
            pltpu.VMEM((_BW,), f32),
            pltpu.VMEM((_BW, _NE), f32),
            pltpu.SemaphoreType.DMA,
        ],
    )
    return kernel_fn(users_id, items_id, envs_id,
                     ue_inv, ub_inv, ie_inv, ib_inv,
                     ue_env, ub_env, ie_env, ib_env,
                     env_emb, env_bias, cls_w, cls_b)


def kernel(users_id, items_id, envs_id, alpha,
           user_emb_inv, user_bias_inv, item_emb_inv, item_bias_inv,
           user_emb_env, user_bias_env, item_emb_env, item_bias_env,
           env_emb, env_bias, cls_W, cls_b):
    del alpha
    inv_score, env_score, env_outputs = _run(
        users_id, items_id, envs_id,
        _linearize(user_emb_inv.T), user_bias_inv.reshape(_U),
        _linearize(item_emb_inv.T), item_bias_inv.reshape(_I),
        _linearize(user_emb_env.T), user_bias_env.reshape(_U),
        _linearize(item_emb_env.T), item_bias_env.reshape(_I),
        env_emb, jnp.pad(env_bias.reshape(_NE), (0, 16 - _NE)),
        cls_W, jnp.pad(cls_b, (0, 16 - _NE)))
    return inv_score, env_score, env_outputs


# speedup vs baseline: 1.9041x; 1.6350x over previous
import functools

import jax
import jax.numpy as jnp
from jax import lax
from jax.experimental import pallas as pl
from jax.experimental.pallas import tpu as pltpu
from jax.experimental.pallas import tpu_sc as plsc

_U = 1000000
_I = 1000000
_NE = 8
_D = 32
_B = 16384

_NC = 2
_NS = 16
_NW = _NC * _NS
_BW = _B // _NW
_CH = 128
_NCH = _BW // _CH
_G = _BW // 16

_LN2 = 0.6931471805599453
_SQRT2 = 1.4142135623730951


def _ln(x):
    xi = lax.bitcast_convert_type(x, jnp.int32)
    e = lax.shift_right_arithmetic(xi, 23) - 127
    m = lax.bitcast_convert_type(
        jnp.bitwise_or(jnp.bitwise_and(xi, 0x007FFFFF), 0x3F800000),
        jnp.float32)
    big = m > _SQRT2
    m = jnp.where(big, m * 0.5, m)
    e = e + jnp.where(big, 1, 0)
    t = (m - 1.0) / (m + 1.0)
    t2 = t * t
    ln_m = t * (2.0 + t2 * (2.0 / 3.0 + t2 * (2.0 / 5.0 + t2 * (2.0 / 7.0))))
    return e.astype(jnp.float32) * _LN2 + ln_m


def _body(users_hbm, items_hbm, envs_hbm,
          ue_inv_hbm, ub_inv_hbm, ie_inv_hbm, ib_inv_hbm,
          ue_env_hbm, ub_env_hbm, ie_env_hbm, ib_env_hbm,
          env_emb_hbm, env_bias_hbm, cls_w_hbm, cls_b_hbm,
          inv_out_hbm, env_out_hbm, eo_out_hbm,
          uidx_v, iidx_v, eidx_v,
          ru_inv, ri_inv, ru_env, ri_env,
          bu_inv, bi_inv, bu_env, bi_env,
          envemb_v, envbias_v, w_v, b_v,
          invs_v, envs_v, eo_v, sem):
    wid = lax.axis_index("s") * _NC + lax.axis_index("c")
    base = wid * _BW

    pltpu.sync_copy(users_hbm.at[pl.ds(base, _BW)], uidx_v)
    pltpu.sync_copy(items_hbm.at[pl.ds(base, _BW)], iidx_v)
    pltpu.sync_copy(envs_hbm.at[pl.ds(base, _BW)], eidx_v)

    copies = []
    for j in range(_NCH):
        s = pl.ds(j * _CH, _CH)
        for tab, idx, dst in (
                (ue_inv_hbm, uidx_v, ru_inv), (ie_inv_hbm, iidx_v, ri_inv),
                (ue_env_hbm, uidx_v, ru_env), (ie_env_hbm, iidx_v, ri_env)):
            copies.append(pltpu.async_copy(tab.at[idx.at[s]], dst.at[s], sem))
        for tab, idx, dst in (
                (ub_inv_hbm, uidx_v, bu_inv), (ib_inv_hbm, iidx_v, bi_inv),
                (ub_env_hbm, uidx_v, bu_env), (ib_env_hbm, iidx_v, bi_env)):
            copies.append(pltpu.async_copy(tab.at[idx.at[s]], dst.at[s], sem))

    pltpu.sync_copy(env_emb_hbm, envemb_v)
    pltpu.sync_copy(env_bias_hbm, envbias_v)
    pltpu.sync_copy(cls_w_hbm, w_v)
    pltpu.sync_copy(cls_b_hbm, b_v)

    for c in copies:
        c.wait()

    lane = lax.iota(jnp.int32, 16)
    w_rows = [[w_v[e, pl.ds(0, 16)], w_v[e, pl.ds(16, 16)]]
              for e in range(_NE)]
    b_vec = plsc.load_gather(b_v, [jnp.bitwise_and(lane, _NE - 1)])

    def group(g, carry):
        b16 = g * 16 + lane
        eids = eidx_v[pl.ds(g * 16, 16)]

        inv_acc = bu_inv[pl.ds(g * 16, 16)] + bi_inv[pl.ds(g * 16, 16)]
        env_acc = (bu_env[pl.ds(g * 16, 16)] + bi_env[pl.ds(g * 16, 16)]
                   + plsc.load_gather(envbias_v, [eids]))
        logits = [None] * _NE
        for d in range(_D):
            dd = jnp.full((16,), d, jnp.int32)
            u = plsc.load_gather(ru_inv, [b16, dd])
            i = plsc.load_gather(ri_inv, [b16, dd])
            p = u * i
            inv_acc = inv_acc + p
            ue = plsc.load_gather(ru_env, [b16, dd])
            ie = plsc.load_gather(ri_env, [b16, dd])
            ee = plsc.load_gather(envemb_v, [eids, dd])
            env_acc = env_acc + ue * ie * ee
            k, r = divmod(d, 16)
            for e in range(_NE):
                pw = p * w_rows[e][k][r]
                logits[e] = pw if logits[e] is None else logits[e] + pw

        logits = [logits[e] + b_vec[e] for e in range(_NE)]
        m = logits[0]
        for e in range(1, _NE):
            m = jnp.maximum(m, logits[e])
        exps = [jnp.exp(logits[e] - m) for e in range(_NE)]
        ssum = exps[0]
        for e in range(1, _NE):
            ssum = ssum + exps[e]
        lse = m + _ln(ssum)

        invs_v[pl.ds(g * 16, 16)] = inv_acc
        envs_v[pl.ds(g * 16, 16)] = inv_acc + env_acc
        for e in range(_NE):
            plsc.store_scatter(eo_v, [b16, jnp.full((16,), e, jnp.int32)],
                               logits[e] - lse)
        return carry

    lax.fori_loop(0, _G, group, 0)

    pltpu.sync_copy(invs_v, inv_out_hbm.at[pl.ds(base, _BW)])
    pltpu.sync_copy(envs_v, env_out_hbm.at[pl.ds(base, _BW)])
    pltpu.sync_copy(eo_v, eo_out_hbm.at[pl.ds(base, _BW)])


@jax.jit
def _run(users_id, items_id, envs_id,
         ue_inv, ub_inv, ie_inv, ib_inv,
         ue_env, ub_env, ie_env, ib_env,
         env_emb, env_bias, cls_w, cls_b):
    mesh = plsc.VectorSubcoreMesh(core_axis_name="c", subcore_axis_name="s",
                                  num_cores=_NC, num_subcores=_NS)
    f32 = jnp.float32
    kernel_fn = pl.kernel(
        _body,
        out_type=(jax.ShapeDtypeStruct((_B,), f32),
                  jax.ShapeDtypeStruct((_B,), f32),
                  jax.ShapeDtypeStruct((_B, _NE), f32)),
        mesh=mesh,
        compiler_params=pltpu.CompilerParams(needs_layout_passes=False,
                                             use_tc_tiling_on_sc=False),
        scratch_types=[
            pltpu.VMEM((_BW,), jnp.int32),
            pltpu.VMEM((_BW,), jnp.int32),
            pltpu.VMEM((_BW,), jnp.int32),
            pltpu.VMEM((_BW, _D), f32),
            pltpu.VMEM((_BW, _D), f32),
            pltpu.VMEM((_BW, _D), f32),
            pltpu.VMEM((_BW, _D), f32),
            pltpu.VMEM((_BW,), f32),
            pltpu.VMEM((_BW,), f32),
            pltpu.VMEM((_BW,), f32),
            pltpu.VMEM((_BW,), f32),
            pltpu.VMEM((_NE, _D), f32),
            pltpu.VMEM((16,), f32),
            pltpu.VMEM((_NE, _D), f32),
            pltpu.VMEM((16,), f32),
            pltpu.VMEM((_BW,), f32),
            pltpu.VMEM((_BW,), f32),
            pltpu.VMEM((_BW, _NE), f32),
            pltpu.SemaphoreType.DMA,
        ],
    )
    return kernel_fn(users_id, items_id, envs_id,
                     ue_inv, ub_inv, ie_inv, ib_inv,
                     ue_env, ub_env, ie_env, ib_env,
                     env_emb, env_bias, cls_w, cls_b)


def kernel(users_id, items_id, envs_id, alpha,
           user_emb_inv, user_bias_inv, item_emb_inv, item_bias_inv,
           user_emb_env, user_bias_env, item_emb_env, item_bias_env,
           env_emb, env_bias, cls_W, cls_b):
    del alpha
    inv_score, env_score, env_outputs = _run(
        users_id, items_id, envs_id,
        user_emb_inv, user_bias_inv.reshape(_U),
        item_emb_inv, item_bias_inv.reshape(_I),
        user_emb_env, user_bias_env.reshape(_U),
        item_emb_env, item_bias_env.reshape(_I),
        env_emb, jnp.pad(env_bias.reshape(_NE), (0, 16 - _NE)),
        cls_W, jnp.pad(cls_b, (0, 16 - _NE)))
    return inv_score, env_score, env_outputs
